# baseline (speedup 1.0000x reference)
import jax
import jax.numpy as jnp
from jax import lax
from jax.experimental import pallas as pl
from jax.experimental.pallas import tpu as pltpu
from jax.experimental.pallas import tpu_sc as plsc

_A = 8
_B = 16384
_T = 32
_NB = 64
_TABLE = _T * _NB * _NB
_NQ = 4
_QTR = _TABLE // _NQ
_QT = _T // _NQ
_NW = 32
_CHUNK = (_A * _B) // _NW
_LANES = 16


def _tile_q_body(s0_hbm, s1_hbm, w_hbm, out_hbm, tbl0, tbl1, s0, s1, acc,
                 plo0, phi0, plo1, phi1, pt0, pt1, sem0, sem1):
    wid = lax.axis_index("s") * 2 + lax.axis_index("c")
    base = wid * _CHUNK
    act = wid // 4
    bufs = (tbl0, tbl1)
    sems = (sem0, sem1)

    def start(q):
        return pltpu.async_copy(
            w_hbm.at[act, pl.ds(q * _QTR, _QTR)], bufs[q % 2], sems[q % 2])

    copies = {0: start(0), 1: start(1)}
    pltpu.sync_copy(s0_hbm.at[pl.ds(base, _CHUNK)], s0)
    pltpu.sync_copy(s1_hbm.at[pl.ds(base, _CHUNK)], s1)

    inv32 = jnp.float32(1.0 / 32.0)

    def switch_index(wv, th):
        e = th - wv
        tc = 33 - (33.0 - e * 32.0).astype(jnp.int32)
        b1 = jnp.where(wv + (tc - 1).astype(jnp.float32) * inv32 >= th, 1, 0)
        b2 = jnp.where(wv + tc.astype(jnp.float32) * inv32 >= th, 1, 0)
        return tc + 1 - b1 - b2

    for q in range(_NQ):
        tbl = bufs[q % 2]
        copies[q].wait()

        @plsc.parallel_loop(0, _CHUNK // _LANES, unroll=6)
        def chunk_body(i, q=q, tbl=tbl):
            o = i * _LANES
            if q == 0:
                w0 = s0[pl.ds(o, _LANES)] * 64.0
                w1 = s1[pl.ds(o, _LANES)] * 64.0
                i00 = w0.astype(jnp.int32)
                i10 = w1.astype(jnp.int32)
                th0 = i00.astype(jnp.float32) + 1.0
                th1 = i10.astype(jnp.float32) + 1.0
                lo0 = i00 * _NB
                hi0 = jnp.minimum(i00 + 1, _NB - 1) * _NB
                lo1 = i10
                hi1 = jnp.minimum(i10 + 1, _NB - 1)
                t00 = switch_index(w0, th0)
                t01 = switch_index(w1, th1)
                plo0[pl.ds(o, _LANES)] = lo0
                phi0[pl.ds(o, _LANES)] = hi0
                plo1[pl.ds(o, _LANES)] = lo1
                phi1[pl.ds(o, _LANES)] = hi1
                pt0[pl.ds(o, _LANES)] = t00
                pt1[pl.ds(o, _LANES)] = t01
            else:
                lo0 = plo0[pl.ds(o, _LANES)]
                hi0 = phi0[pl.ds(o, _LANES)]
                lo1 = plo1[pl.ds(o, _LANES)]
                hi1 = phi1[pl.ds(o, _LANES)]
                t00 = pt0[pl.ds(o, _LANES)]
                t01 = pt1[pl.ds(o, _LANES)]
            a0 = jnp.zeros((_LANES,), jnp.float32)
            a1 = jnp.zeros((_LANES,), jnp.float32)
            for tl in range(_QT):
                tg = q * _QT + tl
                if tg == 0:
                    flat = lo0 + lo1
                else:
                    i0s = jnp.where(t00 <= tg, hi0, lo0)
                    i1 = jnp.where(t01 <= tg, hi1, lo1)
                    flat = i0s + i1
                g = plsc.load_gather(
                    tbl.at[pl.ds(tl * (_NB * _NB), _NB * _NB)], [flat])
                if tl % 2 == 0:
                    a0 = a0 + g
                else:
                    a1 = a1 + g
            a = a0 + a1
            if q == 0:
                acc[pl.ds(o, _LANES)] = a
            else:
                plsc.addupdate(acc.at[pl.ds(o, _LANES)], a)

        if q + 2 < _NQ:
            copies[q + 2] = start(q + 2)

    pltpu.sync_copy(acc, out_hbm.at[pl.ds(base, _CHUNK)])


def kernel(state, weights):
    mesh = plsc.VectorSubcoreMesh(core_axis_name="c", subcore_axis_name="s")
    run = pl.kernel(
        _tile_q_body,
        out_type=jax.ShapeDtypeStruct((_A * _B,), jnp.float32),
        mesh=mesh,
        compiler_params=pltpu.CompilerParams(needs_layout_passes=False),
        scratch_types=[
            pltpu.VMEM((_QTR,), jnp.float32),
            pltpu.VMEM((_QTR,), jnp.float32),
            pltpu.VMEM((_CHUNK,), jnp.float32),
            pltpu.VMEM((_CHUNK,), jnp.float32),
            pltpu.VMEM((_CHUNK,), jnp.float32),
            pltpu.VMEM((_CHUNK,), jnp.int32),
            pltpu.VMEM((_CHUNK,), jnp.int32),
            pltpu.VMEM((_CHUNK,), jnp.int32),
            pltpu.VMEM((_CHUNK,), jnp.int32),
            pltpu.VMEM((_CHUNK,), jnp.int32),
            pltpu.VMEM((_CHUNK,), jnp.int32),
            pltpu.SemaphoreType.DMA,
            pltpu.SemaphoreType.DMA,
        ],
    )
    s0 = state[:, :, 0].reshape(-1)
    s1 = state[:, :, 1].reshape(-1)
    return run(s0, s1, weights)


# speedup vs baseline: 1.1262x; 1.1262x over previous
import jax
import jax.numpy as jnp
from jax import lax
from jax.experimental import pallas as pl
from jax.experimental.pallas import tpu as pltpu
from jax.experimental.pallas import tpu_sc as plsc

_A = 8
_B = 16384
_T = 32
_NB = 64
_TABLE = _T * _NB * _NB
_NQ = 4
_QTR = _TABLE // _NQ
_QT = _T // _NQ
_NW = 32
_CHUNK = (_A * _B) // _NW
_LANES = 16


def _tile_q_body(s0_hbm, s1_hbm, w_hbm, out_hbm, tbl0, tbl1, s0, s1, acc,
                 plo0, phi0, plo1, phi1, pt0, pt1, sem0, sem1):
    wid = lax.axis_index("s") * 2 + lax.axis_index("c")
    base = wid * _CHUNK
    act = wid // 4
    bufs = (tbl0, tbl1)
    sems = (sem0, sem1)

    def start(q):
        return pltpu.async_copy(
            w_hbm.at[act, pl.ds(q * _QTR, _QTR)], bufs[q % 2], sems[q % 2])

    copies = {0: start(0), 1: start(1)}
    pltpu.sync_copy(s0_hbm.at[pl.ds(base, _CHUNK)], s0)
    pltpu.sync_copy(s1_hbm.at[pl.ds(base, _CHUNK)], s1)

    inv32 = jnp.float32(1.0 / 32.0)

    def switch_index(wv, th):
        e = th - wv
        tc = 33 - (33.0 - e * 32.0).astype(jnp.int32)
        b1 = jnp.where(wv + (tc - 1).astype(jnp.float32) * inv32 >= th, 1, 0)
        b2 = jnp.where(wv + tc.astype(jnp.float32) * inv32 >= th, 1, 0)
        return tc + 1 - b1 - b2

    for q in range(_NQ):
        tbl = bufs[q % 2]
        copies[q].wait()

        @plsc.parallel_loop(0, _CHUNK // _LANES, unroll=4)
        def chunk_body(i, q=q, tbl=tbl):
            o = i * _LANES
            if q == 0:
                w0 = s0[pl.ds(o, _LANES)] * 64.0
                w1 = s1[pl.ds(o, _LANES)] * 64.0
                i00 = w0.astype(jnp.int32)
                i10 = w1.astype(jnp.int32)
                th0 = i00.astype(jnp.float32) + 1.0
                th1 = i10.astype(jnp.float32) + 1.0
                lo0 = i00 * _NB
                hi0 = jnp.minimum(i00 + 1, _NB - 1) * _NB
                lo1 = i10
                hi1 = jnp.minimum(i10 + 1, _NB - 1)
                t00 = switch_index(w0, th0)
                t01 = switch_index(w1, th1)
                plo0[pl.ds(o, _LANES)] = lo0
                phi0[pl.ds(o, _LANES)] = hi0
                plo1[pl.ds(o, _LANES)] = lo1
                phi1[pl.ds(o, _LANES)] = hi1
                pt0[pl.ds(o, _LANES)] = t00
                pt1[pl.ds(o, _LANES)] = t01
            else:
                lo0 = plo0[pl.ds(o, _LANES)]
                hi0 = phi0[pl.ds(o, _LANES)]
                lo1 = plo1[pl.ds(o, _LANES)]
                hi1 = phi1[pl.ds(o, _LANES)]
                t00 = pt0[pl.ds(o, _LANES)]
                t01 = pt1[pl.ds(o, _LANES)]
            a0 = jnp.zeros((_LANES,), jnp.float32)
            a1 = jnp.zeros((_LANES,), jnp.float32)
            for tl in range(_QT):
                tg = q * _QT + tl
                if tg == 0:
                    flat = lo0 + lo1
                else:
                    i0s = jnp.where(t00 <= tg, hi0, lo0)
                    i1 = jnp.where(t01 <= tg, hi1, lo1)
                    flat = i0s + i1
                g = plsc.load_gather(
                    tbl.at[pl.ds(tl * (_NB * _NB), _NB * _NB)], [flat])
                if tl % 2 == 0:
                    a0 = a0 + g
                else:
                    a1 = a1 + g
            a = a0 + a1
            if q == 0:
                acc[pl.ds(o, _LANES)] = a
            else:
                plsc.addupdate(acc.at[pl.ds(o, _LANES)], a)

        if q + 2 < _NQ:
            copies[q + 2] = start(q + 2)

    pltpu.sync_copy(acc, out_hbm.at[pl.ds(base, _CHUNK)])


def kernel(state, weights):
    mesh = plsc.VectorSubcoreMesh(core_axis_name="c", subcore_axis_name="s")
    run = pl.kernel(
        _tile_q_body,
        out_type=jax.ShapeDtypeStruct((_A * _B,), jnp.float32),
        mesh=mesh,
        compiler_params=pltpu.CompilerParams(needs_layout_passes=False),
        scratch_types=[
            pltpu.VMEM((_QTR,), jnp.float32),
            pltpu.VMEM((_QTR,), jnp.float32),
            pltpu.VMEM((_CHUNK,), jnp.float32),
            pltpu.VMEM((_CHUNK,), jnp.float32),
            pltpu.VMEM((_CHUNK,), jnp.float32),
            pltpu.VMEM((_CHUNK,), jnp.int32),
            pltpu.VMEM((_CHUNK,), jnp.int32),
            pltpu.VMEM((_CHUNK,), jnp.int32),
            pltpu.VMEM((_CHUNK,), jnp.int32),
            pltpu.VMEM((_CHUNK,), jnp.int32),
            pltpu.VMEM((_CHUNK,), jnp.int32),
            pltpu.SemaphoreType.DMA,
            pltpu.SemaphoreType.DMA,
        ],
    )
    s0 = state[:, :, 0].reshape(-1)
    s1 = state[:, :, 1].reshape(-1)
    return run(s0, s1, weights)
